# baseline (speedup 1.0000x reference)
import functools

import jax
import jax.numpy as jnp
import numpy as np
from jax import lax
from jax.experimental import pallas as pl
from jax.experimental.pallas import tpu as pltpu
from jax.experimental.pallas import tpu_sc as plsc

D_MODEL = 768
D_FF = 3072
N_EXP = 8
T_TOK = 2048
TP = 512
P_PAD = 2 * T_TOK + N_EXP * TP
NT = P_PAD // TP

_NC, _NS = 2, 16
_NW = _NC * _NS
_DTOK = T_TOK // (_NW // 2)
_CTOK = T_TOK // _NW


def _cumsum_rows(a):
    s = 1
    while s < T_TOK:
        shifted = jnp.concatenate(
            [jnp.zeros((s, N_EXP), a.dtype), a[: T_TOK - s, :]], axis=0)
        a = a + shifted
        s *= 2
    return a


def _cumsum_lanes(a):
    s = 1
    while s < N_EXP:
        shifted = jnp.concatenate(
            [jnp.zeros((1, s), a.dtype), a[:, : N_EXP - s]], axis=1)
        a = a + shifted
        s *= 2
    return a


def _routing_body(x_ref, wg_ref, pos0_ref, pos1_ref, w0b_ref, w1b_ref,
                  eot_ref):
    x = x_ref[...]
    wg = wg_ref[...]
    logits = lax.dot_general(x, wg, (((1,), (1,)), ((), ())),
                             preferred_element_type=jnp.float32)
    iota = lax.broadcasted_iota(jnp.int32, (T_TOK, N_EXP), 1)
    m1 = jnp.max(logits, axis=1, keepdims=True)
    a1 = jnp.min(jnp.where(logits == m1, iota, N_EXP), axis=1, keepdims=True)
    l2 = jnp.where(iota == a1, jnp.float32(-1e30), logits)
    m2 = jnp.max(l2, axis=1, keepdims=True)
    a2 = jnp.min(jnp.where(l2 == m2, iota, N_EXP), axis=1, keepdims=True)
    w0 = jax.nn.sigmoid(m1 - m2)
    w1 = 1.0 - w0
    mask0 = (iota == a1).astype(jnp.int32)
    mask1 = (iota == a2).astype(jnp.int32)
    cum0 = _cumsum_rows(mask0)
    cum1 = _cumsum_rows(mask1)
    excl0 = cum0 - mask0
    excl1 = cum1 - mask1
    c0 = cum0[T_TOK - 1:T_TOK, :]
    c1 = cum1[T_TOK - 1:T_TOK, :]
    c = c0 + c1
    pc = ((c + (TP - 1)) // TP) * TP
    gp = _cumsum_lanes(pc) - pc
    rank0 = jnp.sum(mask0 * excl0, axis=1, keepdims=True)
    rank1 = jnp.sum(mask1 * (c0 + excl1), axis=1, keepdims=True)
    pos0_ref[...] = jnp.sum(mask0 * gp, axis=1, keepdims=True) + rank0
    pos1_ref[...] = jnp.sum(mask1 * gp, axis=1, keepdims=True) + rank1
    w0b_ref[...] = jnp.broadcast_to(w0, (T_TOK, 16))
    w1b_ref[...] = jnp.broadcast_to(w1, (T_TOK, 16))
    ends = gp + pc
    tstart = lax.broadcasted_iota(jnp.int32, (NT, N_EXP), 0) * TP
    eot_ref[...] = jnp.sum((ends <= tstart).astype(jnp.int32), axis=1,
                           keepdims=True)


def _routing_call(x_flat, wg):
    return pl.pallas_call(
        _routing_body,
        out_shape=(
            jax.ShapeDtypeStruct((T_TOK, 1), jnp.int32),
            jax.ShapeDtypeStruct((T_TOK, 1), jnp.int32),
            jax.ShapeDtypeStruct((T_TOK, 16), jnp.float32),
            jax.ShapeDtypeStruct((T_TOK, 16), jnp.float32),
            jax.ShapeDtypeStruct((NT, 1), jnp.int32),
        ),
    )(x_flat, wg)


def _sc_mesh():
    return plsc.VectorSubcoreMesh(core_axis_name="c", subcore_axis_name="s",
                                  num_cores=_NC, num_subcores=_NS)


def _dispatch_body(x_hbm, pos_hbm, xs_hbm, rows_v, idx_v, sem):
    wid = lax.axis_index("s") * _NC + lax.axis_index("c")
    tbase = (wid % (_NW // 2)) * _DTOK
    g = pltpu.async_copy(x_hbm.at[pl.ds(tbase, _DTOK), :], rows_v, sem)
    pltpu.sync_copy(pos_hbm.at[pl.ds(wid * _DTOK, _DTOK)], idx_v)
    g.wait()
    pltpu.async_copy(rows_v, xs_hbm.at[idx_v], sem).wait()


@functools.cache
def _dispatch_kernel():
    return pl.kernel(
        _dispatch_body,
        out_type=jax.ShapeDtypeStruct((P_PAD, D_MODEL), jnp.float32),
        mesh=_sc_mesh(),
        scratch_types=[
            pltpu.VMEM((_DTOK, D_MODEL), jnp.float32),
            pltpu.VMEM((_DTOK,), jnp.int32),
            pltpu.SemaphoreType.DMA,
        ],
    )


def _ffn_body(eot_sref, xs_ref, w1_ref, b1_ref, w2_ref, b2_ref, y_ref,
              w1c_ref, w2c_ref):
    j = pl.program_id(0)

    @pl.when(eot_sref[j] < N_EXP)
    def _():
        new_run = jnp.logical_or(j == 0,
                                 eot_sref[j] != eot_sref[jnp.maximum(j - 1, 0)])

        @pl.when(new_run)
        def _():
            w1c_ref[...] = w1_ref[0].astype(jnp.bfloat16)
            w2c_ref[...] = w2_ref[0].astype(jnp.bfloat16)

        xb = xs_ref[...].astype(jnp.bfloat16)
        h = lax.dot_general(xb, w1c_ref[...], (((1,), (1,)), ((), ())),
                            preferred_element_type=jnp.float32)
        h = h + b1_ref[0]
        h = 0.5 * h * (1.0 + lax.erf(h * np.float32(0.7071067811865476)))
        y = lax.dot_general(h.astype(jnp.bfloat16), w2c_ref[...],
                            (((1,), (1,)), ((), ())),
                            preferred_element_type=jnp.float32)
        y_ref[...] = y + b2_ref[0]


def _ffn_call(eot, xs, w1b, b1, w2b, b2):
    grid_spec = pltpu.PrefetchScalarGridSpec(
        num_scalar_prefetch=1,
        grid=(NT,),
        in_specs=[
            pl.BlockSpec((TP, D_MODEL), lambda j, eot: (j, 0)),
            pl.BlockSpec((1, D_FF, D_MODEL),
                         lambda j, eot: (jnp.minimum(eot[j], N_EXP - 1), 0, 0)),
            pl.BlockSpec((1, 1, D_FF),
                         lambda j, eot: (jnp.minimum(eot[j], N_EXP - 1), 0, 0)),
            pl.BlockSpec((1, D_MODEL, D_FF),
                         lambda j, eot: (jnp.minimum(eot[j], N_EXP - 1), 0, 0)),
            pl.BlockSpec((1, 1, D_MODEL),
                         lambda j, eot: (jnp.minimum(eot[j], N_EXP - 1), 0, 0)),
        ],
        out_specs=pl.BlockSpec((TP, D_MODEL), lambda j, eot: (j, 0)),
        scratch_shapes=[
            pltpu.VMEM((D_FF, D_MODEL), jnp.bfloat16),
            pltpu.VMEM((D_MODEL, D_FF), jnp.bfloat16),
        ],
    )
    return pl.pallas_call(
        _ffn_body,
        grid_spec=grid_spec,
        out_shape=jax.ShapeDtypeStruct((P_PAD, D_MODEL), jnp.float32),
    )(eot, xs, w1b, b1, w2b, b2)


def _combine_body(y_hbm, pos0_hbm, pos1_hbm, w0b_hbm, w1b_hbm, out_hbm,
                  i0_v, i1_v, r0_v, r1_v, w0_v, w1_v, sem):
    wid = lax.axis_index("s") * _NC + lax.axis_index("c")
    tbase = wid * _CTOK
    pltpu.sync_copy(pos0_hbm.at[pl.ds(tbase, _CTOK)], i0_v)
    pltpu.sync_copy(pos1_hbm.at[pl.ds(tbase, _CTOK)], i1_v)
    g0 = pltpu.async_copy(y_hbm.at[i0_v], r0_v, sem)
    g1 = pltpu.async_copy(y_hbm.at[i1_v], r1_v, sem)
    pltpu.sync_copy(w0b_hbm.at[pl.ds(tbase, _CTOK), :], w0_v)
    pltpu.sync_copy(w1b_hbm.at[pl.ds(tbase, _CTOK), :], w1_v)
    g0.wait()
    g1.wait()

    def blend_token(t, carry):
        ws0 = w0_v[t, :]
        ws1 = w1_v[t, :]
        for v in range(D_MODEL // 16):
            sl = pl.ds(v * 16, 16)
            r0_v[t, sl] = r0_v[t, sl] * ws0 + r1_v[t, sl] * ws1
        return carry

    lax.fori_loop(0, _CTOK, blend_token, 0)
    pltpu.sync_copy(r0_v, out_hbm.at[pl.ds(tbase, _CTOK), :])


@functools.cache
def _combine_kernel():
    return pl.kernel(
        _combine_body,
        out_type=jax.ShapeDtypeStruct((T_TOK, D_MODEL), jnp.float32),
        mesh=_sc_mesh(),
        scratch_types=[
            pltpu.VMEM((_CTOK,), jnp.int32),
            pltpu.VMEM((_CTOK,), jnp.int32),
            pltpu.VMEM((_CTOK, D_MODEL), jnp.float32),
            pltpu.VMEM((_CTOK, D_MODEL), jnp.float32),
            pltpu.VMEM((_CTOK, 16), jnp.float32),
            pltpu.VMEM((_CTOK, 16), jnp.float32),
            pltpu.SemaphoreType.DMA,
        ],
    )


def kernel(x, Wg, W1, b1, W2, b2):
    B, S, d = x.shape
    x_flat = x.reshape(-1, d)
    pos0, pos1, w0b, w1b, eot = _routing_call(x_flat, Wg)
    pos0 = pos0.reshape(T_TOK)
    pos1 = pos1.reshape(T_TOK)
    eot = eot.reshape(NT)
    xs = _dispatch_kernel()(x_flat, jnp.concatenate([pos0, pos1]))
    y = _ffn_call(eot, xs, W1, b1.reshape(N_EXP, 1, D_FF),
                  W2, b2.reshape(N_EXP, 1, D_MODEL))
    out = _combine_kernel()(y, pos0, pos1, w0b, w1b)
    return out.reshape(B, S, d), 0.0


# speedup vs baseline: 1.0044x; 1.0044x over previous
import functools

import jax
import jax.numpy as jnp
import numpy as np
from jax import lax
from jax.experimental import pallas as pl
from jax.experimental.pallas import tpu as pltpu
from jax.experimental.pallas import tpu_sc as plsc

D_MODEL = 768
D_FF = 3072
N_EXP = 8
T_TOK = 2048
TP = 512
P_PAD = 2 * T_TOK + N_EXP * TP
NT = P_PAD // TP

_NC, _NS = 2, 16
_NW = _NC * _NS
_DTOK = T_TOK // (_NW // 2)
_CTOK = T_TOK // _NW


def _cumsum_rows(a):
    s = 1
    while s < T_TOK:
        shifted = jnp.concatenate(
            [jnp.zeros((s, N_EXP), a.dtype), a[: T_TOK - s, :]], axis=0)
        a = a + shifted
        s *= 2
    return a


def _cumsum_lanes(a):
    s = 1
    while s < N_EXP:
        shifted = jnp.concatenate(
            [jnp.zeros((1, s), a.dtype), a[:, : N_EXP - s]], axis=1)
        a = a + shifted
        s *= 2
    return a


def _routing_body(x_ref, wg_ref, pos0_ref, pos1_ref, w0b_ref, w1b_ref,
                  eot_ref):
    x = x_ref[...]
    wg = wg_ref[...]
    logits = lax.dot_general(x, wg, (((1,), (1,)), ((), ())),
                             preferred_element_type=jnp.float32)
    iota = lax.broadcasted_iota(jnp.int32, (T_TOK, N_EXP), 1)
    m1 = jnp.max(logits, axis=1, keepdims=True)
    a1 = jnp.min(jnp.where(logits == m1, iota, N_EXP), axis=1, keepdims=True)
    l2 = jnp.where(iota == a1, jnp.float32(-1e30), logits)
    m2 = jnp.max(l2, axis=1, keepdims=True)
    a2 = jnp.min(jnp.where(l2 == m2, iota, N_EXP), axis=1, keepdims=True)
    w0 = jax.nn.sigmoid(m1 - m2)
    w1 = 1.0 - w0
    mask0 = (iota == a1).astype(jnp.int32)
    mask1 = (iota == a2).astype(jnp.int32)
    cum0 = _cumsum_rows(mask0)
    cum1 = _cumsum_rows(mask1)
    excl0 = cum0 - mask0
    excl1 = cum1 - mask1
    c0 = cum0[T_TOK - 1:T_TOK, :]
    c1 = cum1[T_TOK - 1:T_TOK, :]
    c = c0 + c1
    pc = ((c + (TP - 1)) // TP) * TP
    gp = _cumsum_lanes(pc) - pc
    rank0 = jnp.sum(mask0 * excl0, axis=1, keepdims=True)
    rank1 = jnp.sum(mask1 * (c0 + excl1), axis=1, keepdims=True)
    pos0_ref[...] = jnp.sum(mask0 * gp, axis=1, keepdims=True) + rank0
    pos1_ref[...] = jnp.sum(mask1 * gp, axis=1, keepdims=True) + rank1
    w0b_ref[...] = jnp.broadcast_to(w0, (T_TOK, 16))
    w1b_ref[...] = jnp.broadcast_to(w1, (T_TOK, 16))
    ends = gp + pc
    tstart = lax.broadcasted_iota(jnp.int32, (NT, N_EXP), 0) * TP
    eot_ref[...] = jnp.sum((ends <= tstart).astype(jnp.int32), axis=1,
                           keepdims=True)


def _routing_call(x_flat, wg):
    return pl.pallas_call(
        _routing_body,
        out_shape=(
            jax.ShapeDtypeStruct((T_TOK, 1), jnp.int32),
            jax.ShapeDtypeStruct((T_TOK, 1), jnp.int32),
            jax.ShapeDtypeStruct((T_TOK, 16), jnp.float32),
            jax.ShapeDtypeStruct((T_TOK, 16), jnp.float32),
            jax.ShapeDtypeStruct((NT, 1), jnp.int32),
        ),
    )(x_flat, wg)


def _sc_mesh():
    return plsc.VectorSubcoreMesh(core_axis_name="c", subcore_axis_name="s",
                                  num_cores=_NC, num_subcores=_NS)


def _dispatch_body(x_hbm, pos_hbm, xs_hbm, rows_v, idx_v, sem):
    wid = lax.axis_index("s") * _NC + lax.axis_index("c")
    tbase = (wid % (_NW // 2)) * _DTOK
    g = pltpu.async_copy(x_hbm.at[pl.ds(tbase, _DTOK), :], rows_v, sem)
    pltpu.sync_copy(pos_hbm.at[pl.ds(wid * _DTOK, _DTOK)], idx_v)
    g.wait()
    pltpu.async_copy(rows_v, xs_hbm.at[idx_v], sem).wait()


@functools.cache
def _dispatch_kernel():
    return pl.kernel(
        _dispatch_body,
        out_type=jax.ShapeDtypeStruct((P_PAD, D_MODEL), jnp.float32),
        mesh=_sc_mesh(),
        scratch_types=[
            pltpu.VMEM((_DTOK, D_MODEL), jnp.float32),
            pltpu.VMEM((_DTOK,), jnp.int32),
            pltpu.SemaphoreType.DMA,
        ],
    )


def _ffn_body(eot_sref, xs_ref, w1_ref, b1_ref, w2_ref, b2_ref, y_ref,
              w1c_ref, w2c_ref):
    j = pl.program_id(0)

    @pl.when(eot_sref[j] < N_EXP)
    def _():
        new_run = jnp.logical_or(j == 0,
                                 eot_sref[j] != eot_sref[jnp.maximum(j - 1, 0)])

        @pl.when(new_run)
        def _():
            w1c_ref[...] = w1_ref[0].astype(jnp.bfloat16)
            w2c_ref[...] = w2_ref[0].astype(jnp.bfloat16)

        xb = xs_ref[...].astype(jnp.bfloat16)
        half = D_FF // 2
        y = b2_ref[0]
        for kc in range(2):
            lo, hi = kc * half, (kc + 1) * half
            h = lax.dot_general(xb, w1c_ref[lo:hi, :],
                                (((1,), (1,)), ((), ())),
                                preferred_element_type=jnp.float32)
            h = h + b1_ref[0][:, lo:hi]
            h = 0.5 * h * (1.0 + lax.erf(h * np.float32(0.7071067811865476)))
            y = y + lax.dot_general(h.astype(jnp.bfloat16), w2c_ref[:, lo:hi],
                                    (((1,), (1,)), ((), ())),
                                    preferred_element_type=jnp.float32)
        y_ref[...] = y


def _ffn_call(eot, xs, w1b, b1, w2b, b2):
    grid_spec = pltpu.PrefetchScalarGridSpec(
        num_scalar_prefetch=1,
        grid=(NT,),
        in_specs=[
            pl.BlockSpec((TP, D_MODEL), lambda j, eot: (j, 0)),
            pl.BlockSpec((1, D_FF, D_MODEL),
                         lambda j, eot: (jnp.minimum(eot[j], N_EXP - 1), 0, 0)),
            pl.BlockSpec((1, 1, D_FF),
                         lambda j, eot: (jnp.minimum(eot[j], N_EXP - 1), 0, 0)),
            pl.BlockSpec((1, D_MODEL, D_FF),
                         lambda j, eot: (jnp.minimum(eot[j], N_EXP - 1), 0, 0)),
            pl.BlockSpec((1, 1, D_MODEL),
                         lambda j, eot: (jnp.minimum(eot[j], N_EXP - 1), 0, 0)),
        ],
        out_specs=pl.BlockSpec((TP, D_MODEL), lambda j, eot: (j, 0)),
        scratch_shapes=[
            pltpu.VMEM((D_FF, D_MODEL), jnp.bfloat16),
            pltpu.VMEM((D_MODEL, D_FF), jnp.bfloat16),
        ],
    )
    return pl.pallas_call(
        _ffn_body,
        grid_spec=grid_spec,
        out_shape=jax.ShapeDtypeStruct((P_PAD, D_MODEL), jnp.float32),
    )(eot, xs, w1b, b1, w2b, b2)


def _combine_body(y_hbm, pos0_hbm, pos1_hbm, w0b_hbm, w1b_hbm, out_hbm,
                  i0_v, i1_v, r0_v, r1_v, w0_v, w1_v, sem):
    wid = lax.axis_index("s") * _NC + lax.axis_index("c")
    tbase = wid * _CTOK
    pltpu.sync_copy(pos0_hbm.at[pl.ds(tbase, _CTOK)], i0_v)
    pltpu.sync_copy(pos1_hbm.at[pl.ds(tbase, _CTOK)], i1_v)
    g0 = pltpu.async_copy(y_hbm.at[i0_v], r0_v, sem)
    g1 = pltpu.async_copy(y_hbm.at[i1_v], r1_v, sem)
    pltpu.sync_copy(w0b_hbm.at[pl.ds(tbase, _CTOK), :], w0_v)
    pltpu.sync_copy(w1b_hbm.at[pl.ds(tbase, _CTOK), :], w1_v)
    g0.wait()
    g1.wait()

    def blend_token(t, carry):
        ws0 = w0_v[t, :]
        ws1 = w1_v[t, :]
        for v in range(D_MODEL // 16):
            sl = pl.ds(v * 16, 16)
            r0_v[t, sl] = r0_v[t, sl] * ws0 + r1_v[t, sl] * ws1
        return carry

    lax.fori_loop(0, _CTOK, blend_token, 0)
    pltpu.sync_copy(r0_v, out_hbm.at[pl.ds(tbase, _CTOK), :])


@functools.cache
def _combine_kernel():
    return pl.kernel(
        _combine_body,
        out_type=jax.ShapeDtypeStruct((T_TOK, D_MODEL), jnp.float32),
        mesh=_sc_mesh(),
        scratch_types=[
            pltpu.VMEM((_CTOK,), jnp.int32),
            pltpu.VMEM((_CTOK,), jnp.int32),
            pltpu.VMEM((_CTOK, D_MODEL), jnp.float32),
            pltpu.VMEM((_CTOK, D_MODEL), jnp.float32),
            pltpu.VMEM((_CTOK, 16), jnp.float32),
            pltpu.VMEM((_CTOK, 16), jnp.float32),
            pltpu.SemaphoreType.DMA,
        ],
    )


def kernel(x, Wg, W1, b1, W2, b2):
    B, S, d = x.shape
    x_flat = x.reshape(-1, d)
    pos0, pos1, w0b, w1b, eot = _routing_call(x_flat, Wg)
    pos0 = pos0.reshape(T_TOK)
    pos1 = pos1.reshape(T_TOK)
    eot = eot.reshape(NT)
    xs = _dispatch_kernel()(x_flat, jnp.concatenate([pos0, pos1]))
    y = _ffn_call(eot, xs, W1, b1.reshape(N_EXP, 1, D_FF),
                  W2, b2.reshape(N_EXP, 1, D_MODEL))
    out = _combine_kernel()(y, pos0, pos1, w0b, w1b)
    return out.reshape(B, S, d), 0.0
